# baseline (speedup 1.0000x reference)
import functools

import jax
import jax.numpy as jnp
from jax import lax
from jax.experimental import pallas as pl
from jax.experimental.pallas import tpu as pltpu
from jax.experimental.pallas import tpu_sc as plsc

NUM_ITEMS = 20000
NUM_USERS = 2048
D = 64
BATCH = 512
S = 20
C_IMP = 0.5

NC = 2
NS = 16
NW = NC * NS
RPW = BATCH // NW
P2 = 32

ROWS = 64
RBLKS = BATCH // ROWS

_sc_mesh = functools.partial(
    plsc.VectorSubcoreMesh, core_axis_name="c", subcore_axis_name="s"
)
_sc_params = pltpu.CompilerParams(needs_layout_passes=False)


NBUF = 6


@functools.partial(
    pl.kernel,
    out_type=(
        jax.ShapeDtypeStruct((BATCH, NUM_ITEMS), jnp.float32),
        jax.ShapeDtypeStruct((BATCH, P2), jnp.float32),
    ),
    mesh=_sc_mesh(),
    scratch_types=[
        pltpu.VMEM((RPW,), jnp.int32),
        pltpu.VMEM((RPW, P2), jnp.int32),
    ] + [pltpu.VMEM((NUM_ITEMS,), jnp.float32) for _ in range(NBUF)] + [
        pltpu.VMEM((RPW, P2), jnp.float32),
    ] + [pltpu.SemaphoreType.DMA for _ in range(2 * NBUF)],
    compiler_params=_sc_params,
)
def _gather_rows(table_hbm, uid_hbm, ids_hbm, beta_hbm, bsel_hbm,
                 uid_s, ids_v, *scratch):
    rows_b = scratch[:NBUF]
    bsel_v = scratch[NBUF]
    isems = scratch[NBUF + 1:NBUF + 1 + NBUF]
    osems = scratch[NBUF + 1 + NBUF:]
    wid = lax.axis_index("s") * NC + lax.axis_index("c")
    base = wid * RPW
    pltpu.sync_copy(uid_hbm.at[pl.ds(base, RPW)], uid_s)
    pltpu.sync_copy(ids_hbm.at[pl.ds(base, RPW)], ids_v)
    uids = uid_s[...]

    def start_in(i):
        pltpu.make_async_copy(
            table_hbm.at[uids[i]], rows_b[i % NBUF], isems[i % NBUF]
        ).start()

    def wait_in(i):
        pltpu.make_async_copy(
            table_hbm.at[uids[i]], rows_b[i % NBUF], isems[i % NBUF]
        ).wait()

    def start_out(i):
        pltpu.make_async_copy(
            rows_b[i % NBUF], beta_hbm.at[base + i], osems[i % NBUF]
        ).start()

    def wait_out(i):
        pltpu.make_async_copy(
            rows_b[i % NBUF], beta_hbm.at[base + i], osems[i % NBUF]
        ).wait()

    for j in range(NBUF - 1):
        start_in(j)
    for i in range(RPW):
        row = rows_b[i % NBUF]
        wait_in(i)
        for k in range(P2 // 16):
            sl = pl.ds(k * 16, 16)
            bsel_v[i, sl] = plsc.load_gather(row, [ids_v[i, sl]])
        for k in range(P2 // 16):
            sl = pl.ds(k * 16, 16)
            plsc.store_scatter(row, [ids_v[i, sl]],
                               jnp.zeros((16,), jnp.float32))
        start_out(i)
        nxt = i + NBUF - 1
        if nxt < RPW:
            if nxt >= NBUF:
                wait_out(nxt - NBUF)
            start_in(nxt)
    for i in range(RPW - NBUF, RPW):
        wait_out(i)
    pltpu.sync_copy(bsel_v, bsel_hbm.at[pl.ds(base, RPW)])


@functools.partial(
    pl.kernel,
    out_type=(),
    mesh=_sc_mesh(),
    scratch_types=[
        pltpu.VMEM((RPW, P2), jnp.int32),
        pltpu.VMEM((RPW, P2), jnp.float32),
        pltpu.VMEM((RPW, P2), jnp.float32),
    ] + [pltpu.VMEM((NUM_ITEMS,), jnp.float32) for _ in range(NBUF)]
      + [pltpu.SemaphoreType.DMA for _ in range(2 * NBUF)],
    compiler_params=_sc_params,
)
def _scatter_fix(ids_hbm, bsel_hbm, dots_hbm, dense_ref,
                 ids_v, bsel_v, dots_v, *scratch):
    rows_b = scratch[:NBUF]
    isems = scratch[NBUF:2 * NBUF]
    osems = scratch[2 * NBUF:]
    wid = lax.axis_index("s") * NC + lax.axis_index("c")
    base = wid * RPW
    pltpu.sync_copy(ids_hbm.at[pl.ds(base, RPW)], ids_v)
    pltpu.sync_copy(bsel_hbm.at[pl.ds(base, RPW)], bsel_v)
    pltpu.sync_copy(dots_hbm.at[pl.ds(base, RPW)], dots_v)

    def start_in(i):
        pltpu.make_async_copy(
            dense_ref.at[base + i], rows_b[i % NBUF], isems[i % NBUF]
        ).start()

    def wait_in(i):
        pltpu.make_async_copy(
            dense_ref.at[base + i], rows_b[i % NBUF], isems[i % NBUF]
        ).wait()

    def start_out(i):
        pltpu.make_async_copy(
            rows_b[i % NBUF], dense_ref.at[base + i], osems[i % NBUF]
        ).start()

    def wait_out(i):
        pltpu.make_async_copy(
            rows_b[i % NBUF], dense_ref.at[base + i], osems[i % NBUF]
        ).wait()

    for j in range(NBUF - 1):
        start_in(j)
    for i in range(RPW):
        row = rows_b[i % NBUF]
        wait_in(i)
        pgs = [
            plsc.load_gather(row, [ids_v[i, pl.ds(k * 16, 16)]])
            for k in range(P2 // 16)
        ]
        for k in range(P2 // 16):
            sl = pl.ds(k * 16, 16)
            v = (1.0 - C_IMP * bsel_v[i, sl]) * pgs[k] + C_IMP * dots_v[i, sl]
            plsc.store_scatter(row, [ids_v[i, sl]], v)
        start_out(i)
        nxt = i + NBUF - 1
        if nxt < RPW:
            if nxt >= NBUF:
                wait_out(nxt - NBUF)
            start_in(nxt)
    for i in range(RPW - NBUF, RPW):
        wait_out(i)


def _tr_body(x_ref, o_ref):
    o_ref[...] = x_ref[...].T


_tr_table = pl.pallas_call(
    _tr_body,
    grid=(pl.cdiv(NUM_ITEMS, 512), pl.cdiv(NUM_USERS, 512)),
    in_specs=[pl.BlockSpec((512, 512), lambda i, j: (i, j))],
    out_specs=pl.BlockSpec((512, 512), lambda i, j: (j, i)),
    out_shape=jax.ShapeDtypeStruct((NUM_USERS, NUM_ITEMS), jnp.float32),
)


def _dense_body(
    beta_ref, p_ref, im_ref, u_ref, w_ref, bvec_ref, bim_ref, validf_ref,
    out_ref, dots_ref,
):
    u = u_ref[...]
    uw = jnp.dot(u, w_ref[...], preferred_element_type=jnp.float32)
    ub = lax.dot_general(
        u, bvec_ref[...], (((1,), (1,)), ((), ())),
        preferred_element_type=jnp.float32,
    )
    hist = lax.dot_general(
        uw.astype(jnp.bfloat16), im_ref[...], (((1,), (0,)), ((), ())),
        preferred_element_type=jnp.float32,
    ) + ub
    beta = beta_ref[...]
    cb = C_IMP * beta
    out_ref[...] = (1.0 - cb) * p_ref[...] + cb * hist
    dots = jnp.sum(bim_ref[...] * u[:, None, :], axis=2)
    vf = validf_ref[...]
    dots_ref[...] = vf * dots + (1.0 - vf) * dots[:, 0:1]


_dense_call = pl.pallas_call(
    _dense_body,
    grid=(RBLKS,),
    in_specs=[
        pl.BlockSpec((ROWS, NUM_ITEMS), lambda r: (r, 0)),
        pl.BlockSpec((ROWS, NUM_ITEMS), lambda r: (r, 0)),
        pl.BlockSpec((D, NUM_ITEMS), lambda r: (0, 0)),
        pl.BlockSpec((ROWS, D), lambda r: (r, 0)),
        pl.BlockSpec((D, D), lambda r: (0, 0)),
        pl.BlockSpec((1, D), lambda r: (0, 0)),
        pl.BlockSpec((ROWS, S, D), lambda r: (r, 0, 0)),
        pl.BlockSpec((ROWS, S), lambda r: (r, 0)),
    ],
    out_specs=[
        pl.BlockSpec((ROWS, NUM_ITEMS), lambda r: (r, 0)),
        pl.BlockSpec((ROWS, S), lambda r: (r, 0)),
    ],
    out_shape=[
        jax.ShapeDtypeStruct((BATCH, NUM_ITEMS), jnp.float32),
        jax.ShapeDtypeStruct((BATCH, S), jnp.float32),
    ],
)


def kernel(items_memory, batch_length, batch_user_id, batch_items_id,
           users_history_items, batch_user_memory, batch_items_memory,
           batch_items_personalized_probability, W, b):
    p = batch_items_personalized_probability
    uid = batch_user_id.astype(jnp.int32)

    valid = jnp.arange(S, dtype=jnp.int32)[None, :] < batch_length[:, None]
    validf = valid.astype(jnp.float32)
    ids_sel = jnp.where(valid, batch_items_id, batch_items_id[:, 0:1])
    ids_pad = jnp.concatenate(
        [ids_sel, jnp.broadcast_to(ids_sel[:, 0:1], (BATCH, P2 - S))], axis=1
    )

    beta, bsel = _gather_rows(_tr_table(users_history_items.T), uid, ids_pad)

    dense, dots_sel = _dense_call(
        beta, p, items_memory.T.astype(jnp.bfloat16),
        batch_user_memory, W, b.reshape(1, D),
        batch_items_memory, validf,
    )

    dots_pad = jnp.concatenate(
        [dots_sel, jnp.broadcast_to(dots_sel[:, 0:1], (BATCH, P2 - S))],
        axis=1,
    )
    dense_ref = jax.new_ref(dense)
    _scatter_fix(ids_pad, bsel, dots_pad, dense_ref)
    return dense_ref[...]


# speedup vs baseline: 1.0983x; 1.0983x over previous
import functools

import jax
import jax.numpy as jnp
from jax import lax
from jax.experimental import pallas as pl
from jax.experimental.pallas import tpu as pltpu
from jax.experimental.pallas import tpu_sc as plsc

NUM_ITEMS = 20000
NUM_USERS = 2048
D = 64
BATCH = 512
S = 20
C_IMP = 0.5

NC = 2
NS = 16
NW = NC * NS
RPW = BATCH // NW
P2 = 32

ROWS = 64
RBLKS = BATCH // ROWS

_sc_mesh = functools.partial(
    plsc.VectorSubcoreMesh, core_axis_name="c", subcore_axis_name="s"
)
_sc_params = pltpu.CompilerParams(needs_layout_passes=False)


NBUF = 6


@functools.partial(
    pl.kernel,
    out_type=(
        jax.ShapeDtypeStruct((BATCH, NUM_ITEMS), jnp.float32),
        jax.ShapeDtypeStruct((BATCH, P2), jnp.float32),
    ),
    mesh=_sc_mesh(),
    scratch_types=[
        pltpu.VMEM((RPW,), jnp.int32),
        pltpu.VMEM((RPW, P2), jnp.int32),
    ] + [pltpu.VMEM((NUM_ITEMS,), jnp.float32) for _ in range(NBUF)] + [
        pltpu.VMEM((RPW, P2), jnp.float32),
    ] + [pltpu.SemaphoreType.DMA for _ in range(2 * NBUF)],
    compiler_params=_sc_params,
)
def _gather_rows(table_hbm, uid_hbm, ids_hbm, beta_hbm, bsel_hbm,
                 uid_s, ids_v, *scratch):
    rows_b = scratch[:NBUF]
    bsel_v = scratch[NBUF]
    isems = scratch[NBUF + 1:NBUF + 1 + NBUF]
    osems = scratch[NBUF + 1 + NBUF:]
    wid = lax.axis_index("s") * NC + lax.axis_index("c")
    base = wid * RPW
    pltpu.sync_copy(uid_hbm.at[pl.ds(base, RPW)], uid_s)
    pltpu.sync_copy(ids_hbm.at[pl.ds(base, RPW)], ids_v)
    uids = uid_s[...]

    def start_in(i):
        pltpu.make_async_copy(
            table_hbm.at[uids[i]], rows_b[i % NBUF], isems[i % NBUF]
        ).start()

    def wait_in(i):
        pltpu.make_async_copy(
            table_hbm.at[uids[i]], rows_b[i % NBUF], isems[i % NBUF]
        ).wait()

    def start_out(i):
        pltpu.make_async_copy(
            rows_b[i % NBUF], beta_hbm.at[base + i], osems[i % NBUF]
        ).start()

    def wait_out(i):
        pltpu.make_async_copy(
            rows_b[i % NBUF], beta_hbm.at[base + i], osems[i % NBUF]
        ).wait()

    for j in range(NBUF - 1):
        start_in(j)
    for i in range(RPW):
        row = rows_b[i % NBUF]
        wait_in(i)
        for k in range(P2 // 16):
            sl = pl.ds(k * 16, 16)
            bsel_v[i, sl] = plsc.load_gather(row, [ids_v[i, sl]])
        for k in range(P2 // 16):
            sl = pl.ds(k * 16, 16)
            plsc.store_scatter(row, [ids_v[i, sl]],
                               jnp.zeros((16,), jnp.float32))
        start_out(i)
        nxt = i + NBUF - 1
        if nxt < RPW:
            if nxt >= NBUF:
                wait_out(nxt - NBUF)
            start_in(nxt)
    for i in range(RPW - NBUF, RPW):
        wait_out(i)
    pltpu.sync_copy(bsel_v, bsel_hbm.at[pl.ds(base, RPW)])


@functools.partial(
    pl.kernel,
    out_type=(),
    mesh=_sc_mesh(),
    scratch_types=[
        pltpu.VMEM((RPW, P2), jnp.int32),
        pltpu.VMEM((RPW, P2), jnp.float32),
        pltpu.VMEM((RPW, P2), jnp.float32),
    ] + [pltpu.VMEM((NUM_ITEMS,), jnp.float32) for _ in range(NBUF)]
      + [pltpu.SemaphoreType.DMA for _ in range(2 * NBUF)],
    compiler_params=_sc_params,
)
def _scatter_fix(ids_hbm, bsel_hbm, dots_hbm, dense_ref,
                 ids_v, bsel_v, dots_v, *scratch):
    rows_b = scratch[:NBUF]
    isems = scratch[NBUF:2 * NBUF]
    osems = scratch[2 * NBUF:]
    wid = lax.axis_index("s") * NC + lax.axis_index("c")
    base = wid * RPW
    pltpu.sync_copy(ids_hbm.at[pl.ds(base, RPW)], ids_v)
    pltpu.sync_copy(bsel_hbm.at[pl.ds(base, RPW)], bsel_v)
    pltpu.sync_copy(dots_hbm.at[pl.ds(base, RPW)], dots_v)

    def start_in(i):
        pltpu.make_async_copy(
            dense_ref.at[base + i], rows_b[i % NBUF], isems[i % NBUF]
        ).start()

    def wait_in(i):
        pltpu.make_async_copy(
            dense_ref.at[base + i], rows_b[i % NBUF], isems[i % NBUF]
        ).wait()

    def start_out(i):
        pltpu.make_async_copy(
            rows_b[i % NBUF], dense_ref.at[base + i], osems[i % NBUF]
        ).start()

    def wait_out(i):
        pltpu.make_async_copy(
            rows_b[i % NBUF], dense_ref.at[base + i], osems[i % NBUF]
        ).wait()

    for j in range(NBUF - 1):
        start_in(j)
    for i in range(RPW):
        row = rows_b[i % NBUF]
        wait_in(i)
        pgs = [
            plsc.load_gather(row, [ids_v[i, pl.ds(k * 16, 16)]])
            for k in range(P2 // 16)
        ]
        for k in range(P2 // 16):
            sl = pl.ds(k * 16, 16)
            v = (1.0 - C_IMP * bsel_v[i, sl]) * pgs[k] + C_IMP * dots_v[i, sl]
            plsc.store_scatter(row, [ids_v[i, sl]], v)
        start_out(i)
        nxt = i + NBUF - 1
        if nxt < RPW:
            if nxt >= NBUF:
                wait_out(nxt - NBUF)
            start_in(nxt)
    for i in range(RPW - NBUF, RPW):
        wait_out(i)


def _dense_body(
    beta_ref, p_ref, im_ref, u_ref, w_ref, bvec_ref, bim_ref, validf_ref,
    out_ref, dots_ref,
):
    u = u_ref[...]
    uw = jnp.dot(u, w_ref[...], preferred_element_type=jnp.float32)
    ub = lax.dot_general(
        u, bvec_ref[...], (((1,), (1,)), ((), ())),
        preferred_element_type=jnp.float32,
    )
    hist = lax.dot_general(
        uw.astype(jnp.bfloat16), im_ref[...], (((1,), (0,)), ((), ())),
        preferred_element_type=jnp.float32,
    ) + ub
    beta = beta_ref[...]
    cb = C_IMP * beta
    out_ref[...] = (1.0 - cb) * p_ref[...] + cb * hist
    dots = jnp.sum(bim_ref[...] * u[:, None, :], axis=2)
    vf = validf_ref[...]
    dots_ref[...] = vf * dots + (1.0 - vf) * dots[:, 0:1]


_dense_call = pl.pallas_call(
    _dense_body,
    grid=(RBLKS,),
    in_specs=[
        pl.BlockSpec((ROWS, NUM_ITEMS), lambda r: (r, 0)),
        pl.BlockSpec((ROWS, NUM_ITEMS), lambda r: (r, 0)),
        pl.BlockSpec((D, NUM_ITEMS), lambda r: (0, 0)),
        pl.BlockSpec((ROWS, D), lambda r: (r, 0)),
        pl.BlockSpec((D, D), lambda r: (0, 0)),
        pl.BlockSpec((1, D), lambda r: (0, 0)),
        pl.BlockSpec((ROWS, S, D), lambda r: (r, 0, 0)),
        pl.BlockSpec((ROWS, S), lambda r: (r, 0)),
    ],
    out_specs=[
        pl.BlockSpec((ROWS, NUM_ITEMS), lambda r: (r, 0)),
        pl.BlockSpec((ROWS, S), lambda r: (r, 0)),
    ],
    out_shape=[
        jax.ShapeDtypeStruct((BATCH, NUM_ITEMS), jnp.float32),
        jax.ShapeDtypeStruct((BATCH, S), jnp.float32),
    ],
)


def kernel(items_memory, batch_length, batch_user_id, batch_items_id,
           users_history_items, batch_user_memory, batch_items_memory,
           batch_items_personalized_probability, W, b):
    p = batch_items_personalized_probability
    uid = batch_user_id.astype(jnp.int32)

    valid = jnp.arange(S, dtype=jnp.int32)[None, :] < batch_length[:, None]
    validf = valid.astype(jnp.float32)
    ids_sel = jnp.where(valid, batch_items_id, batch_items_id[:, 0:1])
    ids_pad = jnp.concatenate(
        [ids_sel, jnp.broadcast_to(ids_sel[:, 0:1], (BATCH, P2 - S))], axis=1
    )

    beta, bsel = _gather_rows(users_history_items, uid, ids_pad)

    dense, dots_sel = _dense_call(
        beta, p, items_memory.T.astype(jnp.bfloat16),
        batch_user_memory, W, b.reshape(1, D),
        batch_items_memory, validf,
    )

    dots_pad = jnp.concatenate(
        [dots_sel, jnp.broadcast_to(dots_sel[:, 0:1], (BATCH, P2 - S))],
        axis=1,
    )
    dense_ref = jax.new_ref(dense)
    _scatter_fix(ids_pad, bsel, dots_pad, dense_ref)
    return dense_ref[...]
